# baseline (speedup 1.0000x reference)
import jax
import jax.numpy as jnp
from jax import lax
from jax.experimental import pallas as pl
from jax.experimental.pallas import tpu as pltpu
from jax.experimental.pallas import tpu_sc as plsc

VOCAB = 1000000
EMBED_DIM = 64
BATCH = 16384
SEQ = 200

NC = 2
NS = 16
NW = NC * NS
LANES = 16

TCOL = 128
NCOL = VOCAB // TCOL
NFULL_LO = NCOL // NW + 1
NFULL_HI = NCOL % NW
TAIL_W = NCOL % NW
PACK_ROWS = VOCAB // 4
OB_STRIDE = 137

ROWS_PER_W = BATCH // NW
NB = 8
IDX_PER_CHUNK = NB * SEQ
STREAM = 100
NSTREAMS = IDX_PER_CHUNK // STREAM
NCHUNKS = ROWS_PER_W // NB
NPAIRS = NCHUNKS // 2
SCALE = 1.0 / SEQ


def _repack_body(tT, tail, out, in0, in1, ob0, ob1, si0, si1, so0, so1):
    wid = lax.axis_index("s") * NC + lax.axis_index("c")
    nfull = jnp.where(wid < NFULL_HI, NFULL_LO, NFULL_LO - 1)
    ins, obs = (in0, in1), (ob0, ob1)
    sis, sos = (si0, si1), (so0, so1)

    lane = lax.iota(jnp.int32, LANES)

    def col_of(t):
        return wid + NW * t

    def start_in(t, b):
        pltpu.async_copy(tT.at[:, pl.ds(TCOL * col_of(t), TCOL)], ins[b],
                         sis[b])

    def wait_in(b):
        pltpu.make_async_copy(tT.at[:, pl.ds(0, TCOL)], ins[b], sis[b]).wait()

    def transpose_block(ib, ob, nvoc):
        for c in range(nvoc // LANES):
            v0 = c * LANES
            vv = lane + v0
            rowv = lax.shift_right_logical(vv, 2)
            colb = lax.shift_left(lax.bitwise_and(vv, 3), 5)
            for p in range(EMBED_DIM // 2):
                ev = ib[2 * p, pl.ds(v0, LANES)]
                od = ib[2 * p + 1, pl.ds(v0, LANES)]
                packed = plsc.pack(ev, od, format=plsc.PackFormat.INTERLEAVED)
                plsc.store_scatter(ob, [rowv, colb + p],
                                   plsc.bitcast(packed, jnp.float32))

    def fire_out(t, b):
        pltpu.async_copy(obs[b].at[:, pl.ds(0, 128)],
                         out.at[pl.ds(32 * col_of(t), 32)], sos[b])

    def wait_out(b):
        pltpu.make_async_copy(obs[b].at[:, pl.ds(0, 128)],
                              out.at[pl.ds(0, 32)], sos[b]).wait()

    start_in(0, 0)

    def pair(p, _):
        t0 = 2 * p
        start_in(t0 + 1, 1)
        wait_in(0)

        @pl.when(p > 0)
        def _():
            wait_out(0)

        transpose_block(ins[0], obs[0], TCOL)
        fire_out(t0, 0)

        @pl.when(t0 + 2 < nfull)
        def _():
            start_in(t0 + 2, 0)

        wait_in(1)

        @pl.when(p > 0)
        def _():
            wait_out(1)

        transpose_block(ins[1], obs[1], TCOL)
        fire_out(t0 + 1, 1)
        return ()

    lax.fori_loop(0, (NFULL_LO - 1) // 2, pair, ())
    wait_out(0)
    wait_out(1)

    @pl.when(wid < NFULL_HI)
    def _():
        wait_in(0)
        transpose_block(ins[0], obs[0], TCOL)
        fire_out(NFULL_LO - 1, 0)
        wait_out(0)

    @pl.when(wid == TAIL_W)
    def _():
        pltpu.sync_copy(tail, in0)
        transpose_block(in0, ob0, VOCAB - TCOL * NCOL)
        pltpu.sync_copy(ob0.at[pl.ds(0, 16), pl.ds(0, 128)],
                        out.at[pl.ds(32 * NCOL, 16)])


def _encoder_body(dseq, table, out, idx0, idx1, rows0, rows1, outst,
                  sem0, sem1):
    wid = lax.axis_index("s") * NC + lax.axis_index("c")
    base_row = wid * ROWS_PER_W

    idx_bufs = (idx0, idx1)
    row_bufs = (rows0, rows1)
    sems = (sem0, sem1)

    def fire(c, buf):
        ib, rb, sem = idx_bufs[buf], row_bufs[buf], sems[buf]
        irow0 = (base_row + c * NB) * (SEQ // STREAM)
        pltpu.sync_copy(dseq.at[pl.ds(irow0, NSTREAMS)], ib)
        for j in range(NSTREAMS):
            pltpu.async_copy(table.at[ib.at[j]],
                             rb.at[pl.ds(j * STREAM, STREAM)], sem)

    def drain(buf):
        rb, sem = row_bufs[buf], sems[buf]
        pltpu.make_async_copy(table.at[pl.ds(0, IDX_PER_CHUNK)], rb, sem).wait()

    lane2 = 2 * lax.iota(jnp.int32, LANES)

    def compute(c, buf):
        rb = row_bufs[buf]
        for b in range(NB):
            rbase = b * SEQ

            def body(j, accs):
                new = []
                for k in range(EMBED_DIM // 32):
                    x = rb[rbase + j, pl.ds(k * LANES, LANES)]
                    xb = plsc.bitcast(x, jnp.bfloat16)
                    ev, od = plsc.unpack(xb,
                                         format=plsc.PackFormat.INTERLEAVED)
                    new.append(accs[2 * k] + ev)
                    new.append(accs[2 * k + 1] + od)
                return tuple(new)

            zero = jnp.zeros((LANES,), jnp.float32)
            accs = lax.fori_loop(0, SEQ, body, (zero,) * (EMBED_DIM // LANES),
                                 unroll=2)
            rowv = jnp.full((LANES,), b, jnp.int32)
            for k in range(EMBED_DIM // 32):
                plsc.store_scatter(outst, [rowv, k * 32 + lane2],
                                   accs[2 * k] * SCALE)
                plsc.store_scatter(outst, [rowv, k * 32 + 1 + lane2],
                                   accs[2 * k + 1] * SCALE)
        pltpu.sync_copy(outst, out.at[pl.ds(base_row + c * NB, NB)])

    fire(0, 0)

    def pair(p, _):
        c0 = 2 * p
        fire(c0 + 1, 1)
        drain(0)
        compute(c0, 0)

        @pl.when(p + 1 < NPAIRS)
        def _():
            fire(c0 + 2, 0)

        drain(1)
        compute(c0 + 1, 1)
        return ()

    lax.fori_loop(0, NPAIRS, pair, ())


@jax.jit
def kernel(drug_seq, emb_table):
    mesh = plsc.VectorSubcoreMesh(core_axis_name="c", subcore_axis_name="s")
    repack = pl.kernel(
        _repack_body,
        out_type=jax.ShapeDtypeStruct((PACK_ROWS, 128), jnp.float32),
        mesh=mesh,
        scratch_types=[
            pltpu.VMEM((EMBED_DIM, TCOL), jnp.float32),
            pltpu.VMEM((EMBED_DIM, TCOL), jnp.float32),
            pltpu.VMEM((32, OB_STRIDE), jnp.float32),
            pltpu.VMEM((32, OB_STRIDE), jnp.float32),
            pltpu.SemaphoreType.DMA,
            pltpu.SemaphoreType.DMA,
            pltpu.SemaphoreType.DMA,
            pltpu.SemaphoreType.DMA,
        ],
        compiler_params=pltpu.CompilerParams(use_tc_tiling_on_sc=True,
                                             needs_layout_passes=False),
    )
    tT = emb_table.T
    tail = jnp.pad(tT[:, TCOL * NCOL:],
                   ((0, 0), (0, TCOL - (VOCAB - TCOL * NCOL))))
    packed = repack(tT, tail)
    t2 = packed.reshape(VOCAB, 32)

    dseq = drug_seq.reshape(BATCH * (SEQ // STREAM), STREAM).astype(jnp.int32)
    lookup = pl.kernel(
        _encoder_body,
        out_type=jax.ShapeDtypeStruct((BATCH, EMBED_DIM), jnp.float32),
        mesh=mesh,
        scratch_types=[
            pltpu.VMEM((NSTREAMS, STREAM), jnp.int32),
            pltpu.VMEM((NSTREAMS, STREAM), jnp.int32),
            pltpu.VMEM((IDX_PER_CHUNK, 32), jnp.float32),
            pltpu.VMEM((IDX_PER_CHUNK, 32), jnp.float32),
            pltpu.VMEM((NB, EMBED_DIM), jnp.float32),
            pltpu.SemaphoreType.DMA,
            pltpu.SemaphoreType.DMA,
        ],
        compiler_params=pltpu.CompilerParams(use_tc_tiling_on_sc=False,
                                             needs_layout_passes=False),
    )
    return lookup(dseq, t2)


# speedup vs baseline: 1.1925x; 1.1925x over previous
import jax
import jax.numpy as jnp
from jax import lax
from jax.experimental import pallas as pl
from jax.experimental.pallas import tpu as pltpu
from jax.experimental.pallas import tpu_sc as plsc

VOCAB = 1000000
EMBED_DIM = 64
PAD_DIM = 128
BATCH = 16384
SEQ = 200

NC = 2
NS = 16
NW = NC * NS
LANES = 16

ROWS_PER_W = BATCH // NW
NB = 4
IDX_PER_CHUNK = NB * SEQ
STREAM = 100
NSTREAMS = IDX_PER_CHUNK // STREAM
NCHUNKS = ROWS_PER_W // NB
NPAIRS = NCHUNKS // 2
SCALE = 1.0 / SEQ


def _encoder_body(dseq, table, out, idx0, idx1, rows0, rows1, outst,
                  sem0, sem1):
    wid = lax.axis_index("s") * NC + lax.axis_index("c")
    base_row = wid * ROWS_PER_W

    idx_bufs = (idx0, idx1)
    row_bufs = (rows0, rows1)
    sems = (sem0, sem1)

    def fire(c, buf):
        ib, rb, sem = idx_bufs[buf], row_bufs[buf], sems[buf]
        irow0 = (base_row + c * NB) * (SEQ // STREAM)
        pltpu.sync_copy(dseq.at[pl.ds(irow0, NSTREAMS)], ib)
        for j in range(NSTREAMS):
            pltpu.async_copy(table.at[ib.at[j]],
                             rb.at[pl.ds(j * STREAM, STREAM)], sem)

    def drain(buf):
        rb, sem = row_bufs[buf], sems[buf]
        pltpu.make_async_copy(table.at[pl.ds(0, IDX_PER_CHUNK)], rb, sem).wait()

    def compute(c, buf):
        rb = row_bufs[buf]
        for b in range(NB):
            rbase = b * SEQ

            def body(j, accs):
                return tuple(
                    acc + rb[rbase + j, pl.ds(k * LANES, LANES)]
                    for k, acc in enumerate(accs)
                )

            zero = jnp.zeros((LANES,), jnp.float32)
            accs = lax.fori_loop(0, SEQ, body, (zero,) * (EMBED_DIM // LANES),
                                 unroll=4)
            for k, acc in enumerate(accs):
                outst[b, pl.ds(k * LANES, LANES)] = acc * SCALE
        pltpu.sync_copy(outst, out.at[pl.ds(base_row + c * NB, NB)])

    fire(0, 0)

    def pair(p, _):
        c0 = 2 * p
        fire(c0 + 1, 1)
        drain(0)
        compute(c0, 0)

        @pl.when(p + 1 < NPAIRS)
        def _():
            fire(c0 + 2, 0)

        drain(1)
        compute(c0 + 1, 1)
        return ()

    lax.fori_loop(0, NPAIRS, pair, ())


@jax.jit
def kernel(drug_seq, emb_table):
    tpad = jnp.pad(emb_table, ((0, 0), (0, PAD_DIM - EMBED_DIM)))
    t2 = tpad.reshape(2 * VOCAB, EMBED_DIM)
    dseq = (drug_seq.astype(jnp.int32) * 2).reshape(
        BATCH * (SEQ // STREAM), STREAM)
    mesh = plsc.VectorSubcoreMesh(core_axis_name="c", subcore_axis_name="s")
    f = pl.kernel(
        _encoder_body,
        out_type=jax.ShapeDtypeStruct((BATCH, EMBED_DIM), jnp.float32),
        mesh=mesh,
        scratch_types=[
            pltpu.VMEM((NSTREAMS, STREAM), jnp.int32),
            pltpu.VMEM((NSTREAMS, STREAM), jnp.int32),
            pltpu.VMEM((IDX_PER_CHUNK, EMBED_DIM), jnp.float32),
            pltpu.VMEM((IDX_PER_CHUNK, EMBED_DIM), jnp.float32),
            pltpu.VMEM((NB, EMBED_DIM), jnp.float32),
            pltpu.SemaphoreType.DMA,
            pltpu.SemaphoreType.DMA,
        ],
        compiler_params=pltpu.CompilerParams(use_tc_tiling_on_sc=False),
    )
    return f(dseq, t2)
